# baseline (speedup 1.0000x reference)
import functools

import jax
import jax.numpy as jnp
from jax.experimental import pallas as pl
from jax.experimental.pallas import tpu as pltpu


def _mm_body(h_ref, w_ref, o_ref):
    o_ref[...] = jnp.dot(h_ref[...], w_ref[...],
                         preferred_element_type=jnp.float32)


@functools.partial(jax.jit, static_argnames=("bm", "bn"))
def _matmul(h, w, bm=256, bn=256):
    n, k = h.shape
    k2, dout = w.shape
    grid = (pl.cdiv(n, bm), pl.cdiv(dout, bn))
    return pl.pallas_call(
        _mm_body,
        grid=grid,
        in_specs=[
            pl.BlockSpec((bm, k), lambda i, j: (i, 0)),
            pl.BlockSpec((k, bn), lambda i, j: (0, j)),
        ],
        out_specs=pl.BlockSpec((bm, bn), lambda i, j: (i, j)),
        out_shape=jax.ShapeDtypeStruct((n, dout), jnp.float32),
    )(h, w)


def _gat_layer(h, src, dst, W, al, ar, b, n):
    feat = _matmul(h, W)
    el = feat @ al
    er = feat @ ar
    e = jax.nn.leaky_relu(el[src] + er[dst], negative_slope=0.2)
    emax = jax.ops.segment_max(e, dst, num_segments=n)
    emax = jnp.where(jnp.isfinite(emax), emax, 0.0)
    ex = jnp.exp(e - emax[dst])
    denom = jax.ops.segment_sum(ex, dst, num_segments=n)
    alpha = ex / denom[dst]
    out = jax.ops.segment_sum(alpha[:, None] * feat[src], dst, num_segments=n)
    return out + b


def kernel(x, edge_index, W0, al0, ar0, b0, W1, al1, ar1, b1, W2, al2, ar2, b2,
           W3, al3, ar3, b3, W4, al4, ar4, b4):
    src = edge_index[0]
    dst = edge_index[1]
    n = x.shape[0]
    params = [(W0, al0, ar0, b0), (W1, al1, ar1, b1), (W2, al2, ar2, b2),
              (W3, al3, ar3, b3), (W4, al4, ar4, b4)]
    h = x
    for i, (W, al, ar, b) in enumerate(params):
        h = _gat_layer(h, src, dst, W, al, ar, b, n)
        if i < len(params) - 1:
            h = jax.nn.elu(h)
    return h


# speedup vs baseline: 2.1447x; 2.1447x over previous
import functools

import jax
import jax.numpy as jnp
from jax import lax
from jax.experimental import pallas as pl
from jax.experimental.pallas import tpu as pltpu
from jax.experimental.pallas import tpu_sc as plsc

N = 10000
NPAD = 10240
E = 160000
NC = 2
NS = 16
NW = NC * NS
EW = 5120
EP = NW * EW
BE = 128
NB = EW // BE
C = 128
NSL = NPAD // NS
NEG_INF = float("-inf")

_MESH = plsc.VectorSubcoreMesh(core_axis_name="c", subcore_axis_name="s")


def _vgather(x, idx):
    return x.at[idx].get(mode="promise_in_bounds")


def _mm_body(h_ref, w_ref, al_ref, ar_ref, f_ref, el_ref, er_ref):
    j = pl.program_id(1)
    acc = jnp.dot(h_ref[...], w_ref[...], preferred_element_type=jnp.float32)
    for t in range(f_ref.shape[0]):
        f_ref[t] = acc[:, t * C:(t + 1) * C]
    elp = jnp.dot(acc, al_ref[...], preferred_element_type=jnp.float32)
    erp = jnp.dot(acc, ar_ref[...], preferred_element_type=jnp.float32)

    @pl.when(j == 0)
    def _():
        el_ref[...] = jnp.zeros_like(el_ref)
        er_ref[...] = jnp.zeros_like(er_ref)

    el_ref[...] += elp
    er_ref[...] += erp


@functools.lru_cache(maxsize=None)
def _make_matmul(din, dout):
    bm = 512
    bn = min(512, dout)
    grid = (NPAD // bm, dout // bn)
    return pl.pallas_call(
        _mm_body,
        grid=grid,
        in_specs=[
            pl.BlockSpec((bm, din), lambda i, j: (i, 0)),
            pl.BlockSpec((din, bn), lambda i, j: (0, j)),
            pl.BlockSpec((bn, 1), lambda i, j: (j, 0)),
            pl.BlockSpec((bn, 1), lambda i, j: (j, 0)),
        ],
        out_specs=[
            pl.BlockSpec((bn // C, bm, C), lambda i, j: (j, i, 0)),
            pl.BlockSpec((bm, 1), lambda i, j: (i, 0)),
            pl.BlockSpec((bm, 1), lambda i, j: (i, 0)),
        ],
        out_shape=[
            jax.ShapeDtypeStruct((dout // C, NPAD, C), jnp.float32),
            jax.ShapeDtypeStruct((NPAD, 1), jnp.float32),
            jax.ShapeDtypeStruct((NPAD, 1), jnp.float32),
        ],
    )


def _comb_body_elu(p0_ref, p1_ref, b_ref, o_ref):
    v = p0_ref[0] + p1_ref[0] + b_ref[0]
    o_ref[...] = jnp.where(v > 0, v, jnp.exp(v) - 1.0)


def _comb_body_lin(p0_ref, p1_ref, b_ref, o_ref):
    o_ref[...] = p0_ref[0] + p1_ref[0] + b_ref[0]


@functools.lru_cache(maxsize=None)
def _make_combine(ncin, elu):
    rb = 1024
    grid = (ncin, NPAD // rb)
    return pl.pallas_call(
        _comb_body_elu if elu else _comb_body_lin,
        grid=grid,
        in_specs=[
            pl.BlockSpec((1, rb, C), lambda k, r: (k, r, 0)),
            pl.BlockSpec((1, rb, C), lambda k, r: (k, r, 0)),
            pl.BlockSpec((1, 1, C), lambda k, r: (k, 0, 0)),
        ],
        out_specs=pl.BlockSpec((rb, C), lambda k, r: (r, k)),
        out_shape=jax.ShapeDtypeStruct((NPAD, ncin * C), jnp.float32),
    )


def _a1_body(el_h, er_h, src_h, dst_h, e_h, emax_h,
             el_v, er_v, src_v, dst_v, e_v, emax_v, tmp_v, red_v, shr):
    c = lax.axis_index("c")
    s = lax.axis_index("s")
    wid = c * NS + s
    pltpu.sync_copy(el_h, el_v)
    pltpu.sync_copy(er_h, er_v)
    pltpu.sync_copy(src_h.at[pl.ds(wid * EW, EW)], src_v)
    pltpu.sync_copy(dst_h.at[pl.ds(wid * EW, EW)], dst_v)

    def init_body(i, _):
        emax_v[pl.ds(i * 16, 16)] = jnp.full((16,), NEG_INF, jnp.float32)
        return 0

    lax.fori_loop(0, NPAD // 16, init_body, 0)

    iota = lax.iota(jnp.int32, 16)

    def edge_body(i, _):
        sl = pl.ds(i * 16, 16)
        s16 = src_v[sl]
        d16 = dst_v[sl]
        ev = plsc.load_gather(el_v, [s16]) + plsc.load_gather(er_v, [d16])
        ev = jnp.where(ev >= 0.0, ev, 0.2 * ev)
        gidx = wid * EW + i * 16 + iota
        ev = jnp.where(gidx < E, ev, NEG_INF)
        e_v[sl] = ev
        ks, vs = plsc.sort_key_val(d16, ev)
        for sft in (1, 2, 4, 8):
            pidx = jnp.maximum(iota - sft, 0)
            kk = _vgather(ks, pidx)
            vv = _vgather(vs, pidx)
            vs = jnp.where(kk == ks, jnp.maximum(vs, vv), vs)
        kn = _vgather(ks, jnp.minimum(iota + 1, 15))
        last = (ks != kn) | (iota == 15)
        old = plsc.load_gather(emax_v, [ks])
        plsc.store_scatter(emax_v, [ks], jnp.maximum(old, vs), mask=last)
        return 0

    lax.fori_loop(0, EW // 16, edge_body, 0)

    pltpu.sync_copy(e_v, e_h.at[pl.ds(wid * EW, EW)])

    pltpu.sync_copy(emax_v, shr.at[s])
    plsc.subcore_barrier()
    pltpu.sync_copy(shr.at[:, pl.ds(s * NSL, NSL)], tmp_v)

    def red_body(i, _):
        sl = pl.ds(i * 16, 16)
        m = tmp_v[0, sl]
        for kk in range(1, NS):
            m = jnp.maximum(m, tmp_v[kk, sl])
        red_v[sl] = m
        return 0

    lax.fori_loop(0, NSL // 16, red_body, 0)
    pltpu.sync_copy(red_v, emax_h.at[c, pl.ds(s * NSL, NSL)])


def _a2_body(dst_h, e_h, emax_h, ex_h, den_h,
             m_v, tmp_n, dst_v, e_v, ex_v, den_v, tmp_v, red_v, shr):
    c = lax.axis_index("c")
    s = lax.axis_index("s")
    wid = c * NS + s
    pltpu.sync_copy(emax_h.at[0], m_v)
    pltpu.sync_copy(emax_h.at[1], tmp_n)

    def mx_body(i, _):
        sl = pl.ds(i * 16, 16)
        m = jnp.maximum(m_v[sl], tmp_n[sl])
        m_v[sl] = jnp.where(m == NEG_INF, 0.0, m)
        return 0

    lax.fori_loop(0, NPAD // 16, mx_body, 0)

    pltpu.sync_copy(dst_h.at[pl.ds(wid * EW, EW)], dst_v)
    pltpu.sync_copy(e_h.at[pl.ds(wid * EW, EW)], e_v)

    def init_body(i, _):
        den_v[pl.ds(i * 16, 16)] = jnp.zeros((16,), jnp.float32)
        return 0

    lax.fori_loop(0, NPAD // 16, init_body, 0)

    iota = lax.iota(jnp.int32, 16)

    def edge_body(i, _):
        sl = pl.ds(i * 16, 16)
        d16 = dst_v[sl]
        e16 = e_v[sl]
        mx = plsc.load_gather(m_v, [d16])
        ex = jnp.exp(e16 - mx)
        ex = jnp.where(e16 == NEG_INF, 0.0, ex)
        ex_v[sl] = ex
        ks, vs = plsc.sort_key_val(d16, ex)
        for sft in (1, 2, 4, 8):
            pidx = jnp.maximum(iota - sft, 0)
            kk = _vgather(ks, pidx)
            vv = _vgather(vs, pidx)
            vs = jnp.where((kk == ks) & (iota >= sft), vs + vv, vs)
        kn = _vgather(ks, jnp.minimum(iota + 1, 15))
        last = (ks != kn) | (iota == 15)
        old = plsc.load_gather(den_v, [ks])
        plsc.store_scatter(den_v, [ks], old + vs, mask=last)
        return 0

    lax.fori_loop(0, EW // 16, edge_body, 0)

    pltpu.sync_copy(ex_v, ex_h.at[pl.ds(wid * EW, EW)])

    pltpu.sync_copy(den_v, shr.at[s])
    plsc.subcore_barrier()
    pltpu.sync_copy(shr.at[:, pl.ds(s * NSL, NSL)], tmp_v)

    def red_body(i, _):
        sl = pl.ds(i * 16, 16)
        m = tmp_v[0, sl]
        for kk in range(1, NS):
            m = m + tmp_v[kk, sl]
        red_v[sl] = m
        return 0

    lax.fori_loop(0, NSL // 16, red_body, 0)
    pltpu.sync_copy(red_v, den_h.at[c, pl.ds(s * NSL, NSL)])


def _b0_body(dst_h, ex_h, den_h, al_h,
             den_v, tmp_n, dst_v, ex_v, al_v):
    c = lax.axis_index("c")
    s = lax.axis_index("s")
    wid = c * NS + s
    pltpu.sync_copy(den_h.at[0], den_v)
    pltpu.sync_copy(den_h.at[1], tmp_n)

    def dn_body(i, _):
        sl = pl.ds(i * 16, 16)
        den_v[sl] = den_v[sl] + tmp_n[sl]
        return 0

    lax.fori_loop(0, NPAD // 16, dn_body, 0)

    pltpu.sync_copy(dst_h.at[pl.ds(wid * EW, EW)], dst_v)
    pltpu.sync_copy(ex_h.at[pl.ds(wid * EW, EW)], ex_v)

    def al_body(i, _):
        sl = pl.ds(i * 16, 16)
        d16 = dst_v[sl]
        ex16 = ex_v[sl]
        dn = plsc.load_gather(den_v, [d16])
        al_v[sl] = jnp.where(ex16 == 0.0, 0.0, ex16 / dn)
        return 0

    lax.fori_loop(0, EW // 16, al_body, 0)
    pltpu.sync_copy(al_v, al_h.at[pl.ds(wid * EW, EW)])


def _b1_body(nc_chunks, src_h, dst_h, al_h, feat_h, out_h,
             src_v2, dst_v2, sidx_v, al_v, g_v, sem, acc_sh):
    c = lax.axis_index("c")
    s = lax.axis_index("s")
    wid = c * NS + s
    pltpu.sync_copy(src_h.at[wid], src_v2)
    pltpu.sync_copy(dst_h.at[wid], dst_v2)
    pltpu.sync_copy(al_h.at[pl.ds(wid * EW, EW)], al_v)

    def chunk_body(k, _):
        def sx_body(i, _):
            r = i // 8
            sl = pl.ds((i % 8) * 16, 16)
            sidx_v[r, sl] = src_v2[r, sl] + k * NPAD
            return 0

        lax.fori_loop(0, EW // 16, sx_body, 0)

        def z_body(r, _):
            for t in range(C // 16):
                g_v[r, pl.ds(t * 16, 16)] = jnp.zeros((16,), jnp.float32)
            return 0

        lax.fori_loop(0, BE, z_body, 0)

        def zz_body(jj, _):
            pltpu.sync_copy(g_v, acc_sh.at[pl.ds(s * NSL + jj * BE, BE)])
            return 0

        lax.fori_loop(0, NSL // BE, zz_body, 0)
        plsc.subcore_barrier()

        def batch_body(b, _):
            pltpu.async_copy(feat_h.at[sidx_v.at[b]], g_v, sem).wait()

            def scale_body(rr, _):
                base = b * BE + rr * 16
                for j in range(16):
                    asp = plsc.load_gather(
                        al_v, [jnp.full((16,), base + j, jnp.int32)])
                    r = rr * 16 + j
                    for t in range(C // 16):
                        sl = pl.ds(t * 16, 16)
                        g_v[r, sl] = g_v[r, sl] * asp
                return 0

            lax.fori_loop(0, BE // 16, scale_body, 0)
            pltpu.sync_copy(g_v, acc_sh.at[dst_v2.at[b]], add=True)
            return 0

        lax.fori_loop(0, NB, batch_body, 0)
        plsc.subcore_barrier()
        pltpu.sync_copy(acc_sh.at[pl.ds(s * NSL, NSL)],
                        out_h.at[c, k, pl.ds(s * NSL, NSL)])
        return 0

    lax.fori_loop(0, nc_chunks, chunk_body, 0)


_F32 = jnp.float32
_I32 = jnp.int32

_a1_call = pl.kernel(
    _a1_body,
    out_type=(
        jax.ShapeDtypeStruct((EP,), _F32),
        jax.ShapeDtypeStruct((NC, NPAD), _F32),
    ),
    mesh=_MESH,
    compiler_params=pltpu.CompilerParams(needs_layout_passes=False),
    scratch_types=[
        pltpu.VMEM((NPAD,), _F32),
        pltpu.VMEM((NPAD,), _F32),
        pltpu.VMEM((EW,), _I32),
        pltpu.VMEM((EW,), _I32),
        pltpu.VMEM((EW,), _F32),
        pltpu.VMEM((NPAD,), _F32),
        pltpu.VMEM((NS, NSL), _F32),
        pltpu.VMEM((NSL,), _F32),
        pltpu.VMEM_SHARED((NS, NPAD), _F32),
    ],
)

_a2_call = pl.kernel(
    _a2_body,
    out_type=(
        jax.ShapeDtypeStruct((EP,), _F32),
        jax.ShapeDtypeStruct((NC, NPAD), _F32),
    ),
    mesh=_MESH,
    compiler_params=pltpu.CompilerParams(needs_layout_passes=False),
    scratch_types=[
        pltpu.VMEM((NPAD,), _F32),
        pltpu.VMEM((NPAD,), _F32),
        pltpu.VMEM((EW,), _I32),
        pltpu.VMEM((EW,), _F32),
        pltpu.VMEM((EW,), _F32),
        pltpu.VMEM((NPAD,), _F32),
        pltpu.VMEM((NS, NSL), _F32),
        pltpu.VMEM((NSL,), _F32),
        pltpu.VMEM_SHARED((NS, NPAD), _F32),
    ],
)


_b0_call = pl.kernel(
    _b0_body,
    out_type=jax.ShapeDtypeStruct((EP,), _F32),
    mesh=_MESH,
    compiler_params=pltpu.CompilerParams(needs_layout_passes=False),
    scratch_types=[
        pltpu.VMEM((NPAD,), _F32),
        pltpu.VMEM((NPAD,), _F32),
        pltpu.VMEM((EW,), _I32),
        pltpu.VMEM((EW,), _F32),
        pltpu.VMEM((EW,), _F32),
    ],
)


@functools.lru_cache(maxsize=None)
def _make_b1(nc_chunks):
    return pl.kernel(
        functools.partial(_b1_body, nc_chunks),
        out_type=jax.ShapeDtypeStruct((NC, nc_chunks, NPAD, C), _F32),
        mesh=_MESH,
        compiler_params=pltpu.CompilerParams(needs_layout_passes=False),
        scratch_types=[
            pltpu.VMEM((NB, BE), _I32),
            pltpu.VMEM((NB, BE), _I32),
            pltpu.VMEM((NB, BE), _I32),
            pltpu.VMEM((EW,), _F32),
            pltpu.VMEM((BE, C), _F32),
            pltpu.SemaphoreType.DMA,
            pltpu.VMEM_SHARED((NPAD, C), _F32),
        ],
    )


def kernel(x, edge_index, W0, al0, ar0, b0, W1, al1, ar1, b1, W2, al2, ar2, b2,
           W3, al3, ar3, b3, W4, al4, ar4, b4):
    src = edge_index[0].astype(jnp.int32)
    dst = edge_index[1].astype(jnp.int32)
    src_p = jnp.pad(src, (0, EP - E))
    dst_p = jnp.pad(dst, (0, EP - E))
    src3 = src_p.reshape(NW, NB, BE)
    dst3 = dst_p.reshape(NW, NB, BE)

    W4p = jnp.pad(W4, ((0, 0), (0, C - 4)))
    al4p = jnp.pad(al4, (0, C - 4))
    ar4p = jnp.pad(ar4, (0, C - 4))
    b4p = jnp.pad(b4, (0, C - 4))

    params = [(W0, al0, ar0, b0), (W1, al1, ar1, b1), (W2, al2, ar2, b2),
              (W3, al3, ar3, b3), (W4p, al4p, ar4p, b4p)]

    h = jnp.pad(x, ((0, NPAD - N), (0, 0)))
    for li, (W, al, ar, b) in enumerate(params):
        din, dout = W.shape
        nc_chunks = dout // C
        featr, el2, er2 = _make_matmul(din, dout)(
            h, W, al.reshape(dout, 1), ar.reshape(dout, 1))
        el = el2.reshape(NPAD)
        er = er2.reshape(NPAD)
        e_arr, emax_p = _a1_call(el, er, src_p, dst_p)
        ex_arr, den_p = _a2_call(dst_p, e_arr, emax_p)
        al_arr = _b0_call(dst_p, ex_arr, den_p)
        feat_flat = featr.reshape(nc_chunks * NPAD, C)
        out_p = _make_b1(nc_chunks)(src3, dst3, al_arr, feat_flat)
        b_r = b.reshape(nc_chunks, 1, C)
        if li < len(params) - 1:
            h = _make_combine(nc_chunks, True)(out_p[0], out_p[1], b_r)
        else:
            out = _make_combine(nc_chunks, False)(out_p[0], out_p[1], b_r)
    return out[:N, :4]


# speedup vs baseline: 2.7223x; 1.2693x over previous
import functools

import jax
import jax.numpy as jnp
from jax import lax
from jax.experimental import pallas as pl
from jax.experimental.pallas import tpu as pltpu
from jax.experimental.pallas import tpu_sc as plsc

N = 10000
NPAD = 10240
E = 160000
NC = 2
NS = 16
NW = NC * NS
EW = 5120
EP = NW * EW
SW = EP // NS
BE = 128
NB = EW // BE
C = 128
NSL = NPAD // NS
NEG_INF = float("-inf")

_MESH = plsc.VectorSubcoreMesh(core_axis_name="c", subcore_axis_name="s")
_F32 = jnp.float32
_I32 = jnp.int32


def _vgather(x, idx):
    return x.at[idx].get(mode="promise_in_bounds")


def _mm_body(h_ref, w_ref, al_ref, ar_ref, f_ref, el_ref, er_ref):
    j = pl.program_id(1)
    acc = jnp.dot(h_ref[...], w_ref[...], preferred_element_type=jnp.float32)
    for t in range(f_ref.shape[0]):
        f_ref[t] = acc[:, t * C:(t + 1) * C]
    elp = jnp.dot(acc, al_ref[...], preferred_element_type=jnp.float32)
    erp = jnp.dot(acc, ar_ref[...], preferred_element_type=jnp.float32)

    @pl.when(j == 0)
    def _():
        el_ref[...] = jnp.zeros_like(el_ref)
        er_ref[...] = jnp.zeros_like(er_ref)

    el_ref[...] += elp
    er_ref[...] += erp


@functools.lru_cache(maxsize=None)
def _make_matmul(din, dout):
    bm = 512
    bn = min(512, dout)
    grid = (NPAD // bm, dout // bn)
    return pl.pallas_call(
        _mm_body,
        grid=grid,
        in_specs=[
            pl.BlockSpec((bm, din), lambda i, j: (i, 0)),
            pl.BlockSpec((din, bn), lambda i, j: (0, j)),
            pl.BlockSpec((bn, 1), lambda i, j: (j, 0)),
            pl.BlockSpec((bn, 1), lambda i, j: (j, 0)),
        ],
        out_specs=[
            pl.BlockSpec((bn // C, bm, C), lambda i, j: (j, i, 0)),
            pl.BlockSpec((bm, 1), lambda i, j: (i, 0)),
            pl.BlockSpec((bm, 1), lambda i, j: (i, 0)),
        ],
        out_shape=[
            jax.ShapeDtypeStruct((dout // C, NPAD, C), jnp.float32),
            jax.ShapeDtypeStruct((NPAD, 1), jnp.float32),
            jax.ShapeDtypeStruct((NPAD, 1), jnp.float32),
        ],
    )


def _comb_body_elu(p0_ref, p1_ref, b_ref, o_ref):
    v = p0_ref[0] + p1_ref[0] + b_ref[0]
    o_ref[...] = jnp.where(v > 0, v, jnp.exp(v) - 1.0)


def _comb_body_lin(p0_ref, p1_ref, b_ref, o_ref):
    o_ref[...] = p0_ref[0] + p1_ref[0] + b_ref[0]


@functools.lru_cache(maxsize=None)
def _make_combine(ncin, elu):
    rb = 1024
    grid = (ncin, NPAD // rb)
    return pl.pallas_call(
        _comb_body_elu if elu else _comb_body_lin,
        grid=grid,
        in_specs=[
            pl.BlockSpec((1, rb, C), lambda k, r: (k, r, 0)),
            pl.BlockSpec((1, rb, C), lambda k, r: (k, r, 0)),
            pl.BlockSpec((1, 1, C), lambda k, r: (k, 0, 0)),
        ],
        out_specs=pl.BlockSpec((rb, C), lambda k, r: (r, k)),
        out_shape=jax.ShapeDtypeStruct((NPAD, ncin * C), jnp.float32),
    )


def _a_body(el_h, er_h, src_h, dst_h, al_h,
            el_v, er_v, src_v, dst_v, emax_v, den_v, tmp_v, red_v, al_v,
            comb, shr):
    c = lax.axis_index("c")
    s = lax.axis_index("s")
    wid = c * NS + s
    pltpu.sync_copy(el_h, el_v)
    pltpu.sync_copy(er_h, er_v)
    pltpu.sync_copy(src_h.at[pl.ds(s * SW, SW)], src_v)
    pltpu.sync_copy(dst_h.at[pl.ds(s * SW, SW)], dst_v)

    iota = lax.iota(jnp.int32, 16)

    def _edge_e(base_off, i):
        sl = pl.ds(i * 16, 16)
        s16 = src_v[sl]
        d16 = dst_v[sl]
        ev = plsc.load_gather(el_v, [s16]) + plsc.load_gather(er_v, [d16])
        ev = jnp.where(ev >= 0.0, ev, 0.2 * ev)
        gidx = base_off + i * 16 + iota
        return jnp.where(gidx < E, ev, NEG_INF), d16

    def init_max(i, _):
        emax_v[pl.ds(i * 16, 16)] = jnp.full((16,), NEG_INF, jnp.float32)
        return 0

    lax.fori_loop(0, NPAD // 16, init_max, 0)

    def pass1(i, _):
        ev, d16 = _edge_e(s * SW, i)
        ks, vs = plsc.sort_key_val(d16, ev)
        for sft in (1, 2, 4, 8):
            pidx = jnp.maximum(iota - sft, 0)
            kk = _vgather(ks, pidx)
            vv = _vgather(vs, pidx)
            vs = jnp.where(kk == ks, jnp.maximum(vs, vv), vs)
        kn = _vgather(ks, jnp.minimum(iota + 1, 15))
        last = (ks != kn) | (iota == 15)
        old = plsc.load_gather(emax_v, [ks])
        plsc.store_scatter(emax_v, [ks], jnp.maximum(old, vs), mask=last)
        return 0

    lax.fori_loop(0, SW // 16, pass1, 0)

    pltpu.sync_copy(emax_v, shr.at[s])
    plsc.subcore_barrier()
    pltpu.sync_copy(shr.at[:, pl.ds(s * NSL, NSL)], tmp_v)

    def redmax(i, _):
        sl = pl.ds(i * 16, 16)
        m = tmp_v[0, sl]
        for kk in range(1, NS):
            m = jnp.maximum(m, tmp_v[kk, sl])
        red_v[sl] = m
        return 0

    lax.fori_loop(0, NSL // 16, redmax, 0)
    pltpu.sync_copy(red_v, comb.at[pl.ds(s * NSL, NSL)])
    plsc.subcore_barrier()
    pltpu.sync_copy(comb, emax_v)

    def fixinf(i, _):
        sl = pl.ds(i * 16, 16)
        m = emax_v[sl]
        emax_v[sl] = jnp.where(m == NEG_INF, 0.0, m)
        return 0

    lax.fori_loop(0, NPAD // 16, fixinf, 0)

    def init_den(i, _):
        den_v[pl.ds(i * 16, 16)] = jnp.zeros((16,), jnp.float32)
        return 0

    lax.fori_loop(0, NPAD // 16, init_den, 0)

    def pass2(i, _):
        ev, d16 = _edge_e(s * SW, i)
        mx = plsc.load_gather(emax_v, [d16])
        ex = jnp.exp(ev - mx)
        ex = jnp.where(ev == NEG_INF, 0.0, ex)
        ks, vs = plsc.sort_key_val(d16, ex)
        for sft in (1, 2, 4, 8):
            pidx = jnp.maximum(iota - sft, 0)
            kk = _vgather(ks, pidx)
            vv = _vgather(vs, pidx)
            vs = jnp.where((kk == ks) & (iota >= sft), vs + vv, vs)
        kn = _vgather(ks, jnp.minimum(iota + 1, 15))
        last = (ks != kn) | (iota == 15)
        old = plsc.load_gather(den_v, [ks])
        plsc.store_scatter(den_v, [ks], old + vs, mask=last)
        return 0

    lax.fori_loop(0, SW // 16, pass2, 0)

    plsc.subcore_barrier()
    pltpu.sync_copy(den_v, shr.at[s])
    plsc.subcore_barrier()
    pltpu.sync_copy(shr.at[:, pl.ds(s * NSL, NSL)], tmp_v)

    def redsum(i, _):
        sl = pl.ds(i * 16, 16)
        m = tmp_v[0, sl]
        for kk in range(1, NS):
            m = m + tmp_v[kk, sl]
        red_v[sl] = m
        return 0

    lax.fori_loop(0, NSL // 16, redsum, 0)
    pltpu.sync_copy(red_v, comb.at[pl.ds(s * NSL, NSL)])
    plsc.subcore_barrier()
    pltpu.sync_copy(comb, den_v)

    pltpu.sync_copy(src_h.at[pl.ds(wid * EW, EW)], src_v.at[pl.ds(0, EW)])
    pltpu.sync_copy(dst_h.at[pl.ds(wid * EW, EW)], dst_v.at[pl.ds(0, EW)])

    def pass3(i, _):
        ev, d16 = _edge_e(wid * EW, i)
        mx = plsc.load_gather(emax_v, [d16])
        ex = jnp.exp(ev - mx)
        ex = jnp.where(ev == NEG_INF, 0.0, ex)
        dn = plsc.load_gather(den_v, [d16])
        al_v[pl.ds(i * 16, 16)] = jnp.where(ex == 0.0, 0.0, ex / dn)
        return 0

    lax.fori_loop(0, EW // 16, pass3, 0)
    pltpu.sync_copy(al_v, al_h.at[pl.ds(wid * EW, EW)])


def _b1_body(nc_chunks, src_h, dst_h, al_h, feat_h, out_h,
             src_v2, dst_v2, al_v, g0, g1, sem0, sem1, acc_sh):
    c = lax.axis_index("c")
    s = lax.axis_index("s")
    wid = c * NS + s
    pltpu.sync_copy(src_h.at[wid], src_v2)
    pltpu.sync_copy(dst_h.at[wid], dst_v2)
    pltpu.sync_copy(al_h.at[pl.ds(wid * EW, EW)], al_v)

    def _scale(g_ref, b):
        def scale_body(rr, _):
            base = b * BE + rr * 16
            for j in range(16):
                asp = plsc.load_gather(
                    al_v, [jnp.full((16,), base + j, jnp.int32)])
                r = rr * 16 + j
                for t in range(C // 16):
                    sl = pl.ds(t * 16, 16)
                    g_ref[r, sl] = g_ref[r, sl] * asp
            return 0

        lax.fori_loop(0, BE // 16, scale_body, 0)

    def chunk_body(k, _):
        def z_body(r, _):
            for t in range(C // 16):
                g0[r, pl.ds(t * 16, 16)] = jnp.zeros((16,), jnp.float32)
            return 0

        lax.fori_loop(0, BE, z_body, 0)

        def zz_body(jj, _):
            pltpu.sync_copy(g0, acc_sh.at[pl.ds(s * NSL + jj * BE, BE)])
            return 0

        lax.fori_loop(0, NSL // BE, zz_body, 0)
        plsc.subcore_barrier()

        pltpu.async_copy(feat_h.at[src_v2.at[0]], g0, sem0)
        pltpu.async_copy(feat_h.at[src_v2.at[1]], g1, sem1)

        def pair_body(b2, _):
            for j, (g_ref, sem) in enumerate(((g0, sem0), (g1, sem1))):
                b = b2 * 2 + j
                pltpu.make_async_copy(feat_h.at[src_v2.at[b]], g_ref,
                                      sem).wait()
                _scale(g_ref, b)
                pltpu.sync_copy(g_ref, acc_sh.at[dst_v2.at[b]], add=True)

                @pl.when(b + 2 < NB)
                def _():
                    pltpu.async_copy(feat_h.at[src_v2.at[b + 2]], g_ref, sem)

            return 0

        lax.fori_loop(0, NB // 2, pair_body, 0)
        plsc.subcore_barrier()
        pltpu.sync_copy(acc_sh.at[pl.ds(s * NSL, NSL)],
                        out_h.at[c, k, pl.ds(s * NSL, NSL)])

        def sx_body(i, _):
            r = i // 8
            sl = pl.ds((i % 8) * 16, 16)
            src_v2[r, sl] = src_v2[r, sl] + NPAD
            return 0

        lax.fori_loop(0, EW // 16, sx_body, 0)
        return 0

    lax.fori_loop(0, nc_chunks, chunk_body, 0)


_a_call = pl.kernel(
    _a_body,
    out_type=jax.ShapeDtypeStruct((EP,), _F32),
    mesh=_MESH,
    compiler_params=pltpu.CompilerParams(needs_layout_passes=False),
    scratch_types=[
        pltpu.VMEM((NPAD,), _F32),
        pltpu.VMEM((NPAD,), _F32),
        pltpu.VMEM((SW,), _I32),
        pltpu.VMEM((SW,), _I32),
        pltpu.VMEM((NPAD,), _F32),
        pltpu.VMEM((NPAD,), _F32),
        pltpu.VMEM((NS, NSL), _F32),
        pltpu.VMEM((NSL,), _F32),
        pltpu.VMEM((EW,), _F32),
        pltpu.VMEM_SHARED((NPAD,), _F32),
        pltpu.VMEM_SHARED((NS, NPAD), _F32),
    ],
)


@functools.lru_cache(maxsize=None)
def _make_b1(nc_chunks):
    return pl.kernel(
        functools.partial(_b1_body, nc_chunks),
        out_type=jax.ShapeDtypeStruct((NC, nc_chunks, NPAD, C), _F32),
        mesh=_MESH,
        compiler_params=pltpu.CompilerParams(needs_layout_passes=False),
        scratch_types=[
            pltpu.VMEM((NB, BE), _I32),
            pltpu.VMEM((NB, BE), _I32),
            pltpu.VMEM((EW,), _F32),
            pltpu.VMEM((BE, C), _F32),
            pltpu.VMEM((BE, C), _F32),
            pltpu.SemaphoreType.DMA,
            pltpu.SemaphoreType.DMA,
            pltpu.VMEM_SHARED((NPAD, C), _F32),
        ],
    )


def kernel(x, edge_index, W0, al0, ar0, b0, W1, al1, ar1, b1, W2, al2, ar2, b2,
           W3, al3, ar3, b3, W4, al4, ar4, b4):
    src = edge_index[0].astype(jnp.int32)
    dst = edge_index[1].astype(jnp.int32)
    src_p = jnp.pad(src, (0, EP - E))
    dst_p = jnp.pad(dst, (0, EP - E))
    src3 = src_p.reshape(NW, NB, BE)
    dst3 = dst_p.reshape(NW, NB, BE)

    W4p = jnp.pad(W4, ((0, 0), (0, C - 4)))
    al4p = jnp.pad(al4, (0, C - 4))
    ar4p = jnp.pad(ar4, (0, C - 4))
    b4p = jnp.pad(b4, (0, C - 4))

    params = [(W0, al0, ar0, b0), (W1, al1, ar1, b1), (W2, al2, ar2, b2),
              (W3, al3, ar3, b3), (W4p, al4p, ar4p, b4p)]

    h = jnp.pad(x, ((0, NPAD - N), (0, 0)))
    for li, (W, al, ar, b) in enumerate(params):
        din, dout = W.shape
        nc_chunks = dout // C
        featr, el2, er2 = _make_matmul(din, dout)(
            h, W, al.reshape(dout, 1), ar.reshape(dout, 1))
        el = el2.reshape(NPAD)
        er = er2.reshape(NPAD)
        al_arr = _a_call(el, er, src_p, dst_p)
        feat_flat = featr.reshape(nc_chunks * NPAD, C)
        out_p = _make_b1(nc_chunks)(src3, dst3, al_arr, feat_flat)
        b_r = b.reshape(nc_chunks, 1, C)
        if li < len(params) - 1:
            h = _make_combine(nc_chunks, True)(out_p[0], out_p[1], b_r)
        else:
            out = _make_combine(nc_chunks, False)(out_p[0], out_p[1], b_r)
    return out[:N, :4]


# speedup vs baseline: 3.2733x; 1.2024x over previous
import functools

import jax
import jax.numpy as jnp
from jax import lax
from jax.experimental import pallas as pl
from jax.experimental.pallas import tpu as pltpu
from jax.experimental.pallas import tpu_sc as plsc

N = 10000
NPAD = 10240
E = 160000
NC = 2
NS = 16
EP = 163840
SW = EP // NS
BE = 128
NBP = SW // BE
NB0 = 56
NB1 = NBP - NB0
NBMAX = max(NB0, NB1)
EPX = EP + NBMAX * BE
C = 128
NSL = NPAD // NS
NEG_INF = float("-inf")

_MESH = plsc.VectorSubcoreMesh(core_axis_name="c", subcore_axis_name="s")
_F32 = jnp.float32
_I32 = jnp.int32


def _vgather(x, idx):
    return x.at[idx].get(mode="promise_in_bounds")


def _mm_body(h_ref, w_ref, al_ref, ar_ref, f_ref, el_ref, er_ref):
    j = pl.program_id(1)
    acc = jnp.dot(h_ref[...], w_ref[...], preferred_element_type=jnp.float32)
    for t in range(f_ref.shape[0]):
        f_ref[t] = acc[:, t * C:(t + 1) * C]
    elp = jnp.dot(acc, al_ref[...], preferred_element_type=jnp.float32)
    erp = jnp.dot(acc, ar_ref[...], preferred_element_type=jnp.float32)

    @pl.when(j == 0)
    def _():
        el_ref[...] = jnp.zeros_like(el_ref)
        er_ref[...] = jnp.zeros_like(er_ref)

    el_ref[...] += elp
    er_ref[...] += erp


@functools.lru_cache(maxsize=None)
def _make_matmul(din, dout):
    bm = 512
    bn = min(512, dout)
    grid = (NPAD // bm, dout // bn)
    return pl.pallas_call(
        _mm_body,
        grid=grid,
        in_specs=[
            pl.BlockSpec((bm, din), lambda i, j: (i, 0)),
            pl.BlockSpec((din, bn), lambda i, j: (0, j)),
            pl.BlockSpec((bn, 1), lambda i, j: (j, 0)),
            pl.BlockSpec((bn, 1), lambda i, j: (j, 0)),
        ],
        out_specs=[
            pl.BlockSpec((bn // C, bm, C), lambda i, j: (j, i, 0)),
            pl.BlockSpec((bm, 1), lambda i, j: (i, 0)),
            pl.BlockSpec((bm, 1), lambda i, j: (i, 0)),
        ],
        out_shape=[
            jax.ShapeDtypeStruct((dout // C, NPAD, C), jnp.float32),
            jax.ShapeDtypeStruct((NPAD, 1), jnp.float32),
            jax.ShapeDtypeStruct((NPAD, 1), jnp.float32),
        ],
    )


def _comb_body_elu(p0_ref, p1_ref, b_ref, o_ref):
    v = p0_ref[0] + p1_ref[0] + b_ref[0]
    o_ref[...] = jnp.where(v > 0, v, jnp.exp(v) - 1.0)


def _comb_body_lin(p0_ref, p1_ref, b_ref, o_ref):
    o_ref[...] = p0_ref[0] + p1_ref[0] + b_ref[0]


@functools.lru_cache(maxsize=None)
def _make_combine(ncin, elu):
    rb = 1024
    grid = (ncin, NPAD // rb)
    return pl.pallas_call(
        _comb_body_elu if elu else _comb_body_lin,
        grid=grid,
        in_specs=[
            pl.BlockSpec((1, rb, C), lambda k, r: (k, r, 0)),
            pl.BlockSpec((1, rb, C), lambda k, r: (k, r, 0)),
            pl.BlockSpec((1, 1, C), lambda k, r: (k, 0, 0)),
        ],
        out_specs=pl.BlockSpec((rb, C), lambda k, r: (r, k)),
        out_shape=jax.ShapeDtypeStruct((NPAD, ncin * C), jnp.float32),
    )


def _a_body(el_h, er_h, src_h, dst_h, al_h,
            el_v, er_v, src_v, dst_v, emax_v, den_v, tmp_v, red_v, al_v,
            comb, shr):
    c = lax.axis_index("c")
    s = lax.axis_index("s")
    pltpu.sync_copy(el_h, el_v)
    pltpu.sync_copy(er_h, er_v)
    pltpu.sync_copy(src_h.at[pl.ds(s * SW, SW)], src_v)
    pltpu.sync_copy(dst_h.at[pl.ds(s * SW, SW)], dst_v)

    iota = lax.iota(jnp.int32, 16)

    def _edge_e(i):
        sl = pl.ds(i * 16, 16)
        s16 = src_v[sl]
        d16 = dst_v[sl]
        ev = plsc.load_gather(el_v, [s16]) + plsc.load_gather(er_v, [d16])
        ev = jnp.where(ev >= 0.0, ev, 0.2 * ev)
        gidx = s * SW + i * 16 + iota
        return jnp.where(gidx < E, ev, NEG_INF), d16

    def init_max(i, _):
        emax_v[pl.ds(i * 16, 16)] = jnp.full((16,), NEG_INF, jnp.float32)
        return 0

    lax.fori_loop(0, NPAD // 16, init_max, 0)

    def pass1(i, _):
        ev, d16 = _edge_e(i)
        ks, vs = plsc.sort_key_val(d16, ev)
        for sft in (1, 2, 4, 8):
            pidx = jnp.maximum(iota - sft, 0)
            kk = _vgather(ks, pidx)
            vv = _vgather(vs, pidx)
            vs = jnp.where(kk == ks, jnp.maximum(vs, vv), vs)
        kn = _vgather(ks, jnp.minimum(iota + 1, 15))
        last = (ks != kn) | (iota == 15)
        old = plsc.load_gather(emax_v, [ks])
        plsc.store_scatter(emax_v, [ks], jnp.maximum(old, vs), mask=last)
        return 0

    lax.fori_loop(0, SW // 16, pass1, 0)

    pltpu.sync_copy(emax_v, shr.at[s])
    plsc.subcore_barrier()
    pltpu.sync_copy(shr.at[:, pl.ds(s * NSL, NSL)], tmp_v)

    def redmax(i, _):
        sl = pl.ds(i * 16, 16)
        m = tmp_v[0, sl]
        for kk in range(1, NS):
            m = jnp.maximum(m, tmp_v[kk, sl])
        red_v[sl] = m
        return 0

    lax.fori_loop(0, NSL // 16, redmax, 0)
    pltpu.sync_copy(red_v, comb.at[pl.ds(s * NSL, NSL)])
    plsc.subcore_barrier()
    pltpu.sync_copy(comb, emax_v)

    def fixinf(i, _):
        sl = pl.ds(i * 16, 16)
        m = emax_v[sl]
        emax_v[sl] = jnp.where(m == NEG_INF, 0.0, m)
        return 0

    lax.fori_loop(0, NPAD // 16, fixinf, 0)

    def init_den(i, _):
        den_v[pl.ds(i * 16, 16)] = jnp.zeros((16,), jnp.float32)
        return 0

    lax.fori_loop(0, NPAD // 16, init_den, 0)

    def pass2(i, _):
        ev, d16 = _edge_e(i)
        mx = plsc.load_gather(emax_v, [d16])
        ex = jnp.exp(ev - mx)
        ex = jnp.where(ev == NEG_INF, 0.0, ex)
        ks, vs = plsc.sort_key_val(d16, ex)
        for sft in (1, 2, 4, 8):
            pidx = jnp.maximum(iota - sft, 0)
            kk = _vgather(ks, pidx)
            vv = _vgather(vs, pidx)
            vs = jnp.where((kk == ks) & (iota >= sft), vs + vv, vs)
        kn = _vgather(ks, jnp.minimum(iota + 1, 15))
        last = (ks != kn) | (iota == 15)
        old = plsc.load_gather(den_v, [ks])
        plsc.store_scatter(den_v, [ks], old + vs, mask=last)
        return 0

    lax.fori_loop(0, SW // 16, pass2, 0)

    plsc.subcore_barrier()
    pltpu.sync_copy(den_v, shr.at[s])
    plsc.subcore_barrier()
    pltpu.sync_copy(shr.at[:, pl.ds(s * NSL, NSL)], tmp_v)

    def redsum(i, _):
        sl = pl.ds(i * 16, 16)
        m = tmp_v[0, sl]
        for kk in range(1, NS):
            m = m + tmp_v[kk, sl]
        red_v[sl] = m
        return 0

    lax.fori_loop(0, NSL // 16, redsum, 0)
    pltpu.sync_copy(red_v, comb.at[pl.ds(s * NSL, NSL)])
    plsc.subcore_barrier()
    pltpu.sync_copy(comb, den_v)

    def pass3(i, _):
        ev, d16 = _edge_e(i)
        mx = plsc.load_gather(emax_v, [d16])
        ex = jnp.exp(ev - mx)
        ex = jnp.where(ev == NEG_INF, 0.0, ex)
        dn = plsc.load_gather(den_v, [d16])
        al_v[pl.ds(i * 16, 16)] = jnp.where(ex == 0.0, 0.0, ex / dn)
        return 0

    lax.fori_loop(0, SW // 16, pass3, 0)
    half = SW // 2
    pltpu.sync_copy(al_v.at[pl.ds(c * half, half)],
                    al_h.at[pl.ds(s * SW + c * half, half)])


def _b1_body(nc_chunks, src2_h, dst2_h, al_h, feat_h, out_h,
             src_v2, dst_v2, a0, a1, g0, g1, sem0, sem1, acc_sh):
    c = lax.axis_index("c")
    s = lax.axis_index("s")
    row0 = pl.multiple_of(s * NBP + c * NB0, 8)
    nb = jnp.where(c == 0, NB0, NB1)
    ebase = row0 * BE
    pltpu.sync_copy(src2_h.at[pl.ds(row0, NBMAX)], src_v2)
    pltpu.sync_copy(dst2_h.at[pl.ds(row0, NBMAX)], dst_v2)

    def _gather(b, g_ref, a_ref, sem):
        pltpu.async_copy(feat_h.at[src_v2.at[b]], g_ref, sem)
        pltpu.async_copy(al_h.at[pl.ds(ebase + b * BE, BE)], a_ref, sem)

    def _wait(b, g_ref, a_ref, sem):
        pltpu.make_async_copy(feat_h.at[src_v2.at[b]], g_ref, sem).wait()
        pltpu.make_async_copy(al_h.at[pl.ds(ebase + b * BE, BE)], a_ref,
                              sem).wait()

    def _scale(g_ref, a_ref):
        def scale_body(rr, _):
            for j in range(16):
                asp = plsc.load_gather(
                    a_ref, [jnp.full((16,), rr * 16 + j, jnp.int32)])
                r = rr * 16 + j
                for t in range(C // 16):
                    sl = pl.ds(t * 16, 16)
                    g_ref[r, sl] = g_ref[r, sl] * asp
            return 0

        lax.fori_loop(0, BE // 16, scale_body, 0)

    def chunk_body(k, _):
        def z_body(r, _):
            for t in range(C // 16):
                g0[r, pl.ds(t * 16, 16)] = jnp.zeros((16,), jnp.float32)
            return 0

        lax.fori_loop(0, BE, z_body, 0)

        def zz_body(jj, _):
            pltpu.sync_copy(g0, acc_sh.at[pl.ds(s * NSL + jj * BE, BE)])
            return 0

        lax.fori_loop(0, NSL // BE, zz_body, 0)
        plsc.subcore_barrier()

        _gather(0, g0, a0, sem0)
        _gather(1, g1, a1, sem1)

        def pair_body(b2, _):
            for j, (g_ref, a_ref, sem) in enumerate(
                    ((g0, a0, sem0), (g1, a1, sem1))):
                b = b2 * 2 + j
                _wait(b, g_ref, a_ref, sem)
                _scale(g_ref, a_ref)
                pltpu.sync_copy(g_ref, acc_sh.at[dst_v2.at[b]], add=True)

                @pl.when(b + 2 < nb)
                def _():
                    _gather(b + 2, g_ref, a_ref, sem)

            return 0

        lax.fori_loop(0, nb // 2, pair_body, 0)
        plsc.subcore_barrier()
        pltpu.sync_copy(acc_sh.at[pl.ds(s * NSL, NSL)],
                        out_h.at[c, k, pl.ds(s * NSL, NSL)])

        def sx_body(i, _):
            r = i // 8
            sl = pl.ds((i % 8) * 16, 16)
            src_v2[r, sl] = src_v2[r, sl] + NPAD
            return 0

        lax.fori_loop(0, NBMAX * BE // 16, sx_body, 0)
        return 0

    lax.fori_loop(0, nc_chunks, chunk_body, 0)


_a_call = pl.kernel(
    _a_body,
    out_type=jax.ShapeDtypeStruct((EP,), _F32),
    mesh=_MESH,
    compiler_params=pltpu.CompilerParams(needs_layout_passes=False),
    scratch_types=[
        pltpu.VMEM((NPAD,), _F32),
        pltpu.VMEM((NPAD,), _F32),
        pltpu.VMEM((SW,), _I32),
        pltpu.VMEM((SW,), _I32),
        pltpu.VMEM((NPAD,), _F32),
        pltpu.VMEM((NPAD,), _F32),
        pltpu.VMEM((NS, NSL), _F32),
        pltpu.VMEM((NSL,), _F32),
        pltpu.VMEM((SW,), _F32),
        pltpu.VMEM_SHARED((NPAD,), _F32),
        pltpu.VMEM_SHARED((NS, NPAD), _F32),
    ],
)


@functools.lru_cache(maxsize=None)
def _make_b1(nc_chunks):
    return pl.kernel(
        functools.partial(_b1_body, nc_chunks),
        out_type=jax.ShapeDtypeStruct((NC, nc_chunks, NPAD, C), _F32),
        mesh=_MESH,
        compiler_params=pltpu.CompilerParams(needs_layout_passes=False),
        scratch_types=[
            pltpu.VMEM((NBMAX, BE), _I32),
            pltpu.VMEM((NBMAX, BE), _I32),
            pltpu.VMEM((BE,), _F32),
            pltpu.VMEM((BE,), _F32),
            pltpu.VMEM((BE, C), _F32),
            pltpu.VMEM((BE, C), _F32),
            pltpu.SemaphoreType.DMA,
            pltpu.SemaphoreType.DMA,
            pltpu.VMEM_SHARED((NPAD, C), _F32),
        ],
    )


def kernel(x, edge_index, W0, al0, ar0, b0, W1, al1, ar1, b1, W2, al2, ar2, b2,
           W3, al3, ar3, b3, W4, al4, ar4, b4):
    src = edge_index[0].astype(jnp.int32)
    dst = edge_index[1].astype(jnp.int32)
    src_p = jnp.pad(src, (0, EP - E))
    dst_p = jnp.pad(dst, (0, EP - E))
    src_px = jnp.pad(src, (0, EPX - E))
    dst_px = jnp.pad(dst, (0, EPX - E))
    src2 = src_px.reshape(EPX // BE, BE)
    dst2 = dst_px.reshape(EPX // BE, BE)

    W4p = jnp.pad(W4, ((0, 0), (0, C - 4)))
    al4p = jnp.pad(al4, (0, C - 4))
    ar4p = jnp.pad(ar4, (0, C - 4))
    b4p = jnp.pad(b4, (0, C - 4))

    params = [(W0, al0, ar0, b0), (W1, al1, ar1, b1), (W2, al2, ar2, b2),
              (W3, al3, ar3, b3), (W4p, al4p, ar4p, b4p)]

    h = jnp.pad(x, ((0, NPAD - N), (0, 0)))
    for li, (W, al, ar, b) in enumerate(params):
        din, dout = W.shape
        nc_chunks = dout // C
        featr, el2, er2 = _make_matmul(din, dout)(
            h, W, al.reshape(dout, 1), ar.reshape(dout, 1))
        el = el2.reshape(NPAD)
        er = er2.reshape(NPAD)
        al_arr = _a_call(el, er, src_p, dst_p)
        feat_flat = featr.reshape(nc_chunks * NPAD, C)
        out_p = _make_b1(nc_chunks)(src2, dst2, al_arr, feat_flat)
        b_r = b.reshape(nc_chunks, 1, C)
        if li < len(params) - 1:
            h = _make_combine(nc_chunks, True)(out_p[0], out_p[1], b_r)
        else:
            out = _make_combine(nc_chunks, False)(out_p[0], out_p[1], b_r)
    return out[:N, :4]
